# baseline (speedup 1.0000x reference)
import functools

import jax
import jax.numpy as jnp
import numpy as np
from jax import lax
from jax.experimental import pallas as pl
from jax.experimental.pallas import tpu as pltpu
from jax.experimental.pallas import tpu_sc as plsc

NUM_THETAS = 32
BUMP_STEPS = 32
NUM_FEATURES = 128
R = 1.1
SCALE = 8.0
NG = 64
BLK = 12544
ST = BUMP_STEPS * NUM_THETAS

_LIN = np.linspace(-R, R, BUMP_STEPS, dtype=np.float32)
_E = [float(v) for v in np.exp(np.float64(SCALE) * _LIN)]


def _fused(batch_ref, x_ref, vs_ref, out_ref):
    i = pl.program_id(0)

    @pl.when(i == 0)
    def _init():
        out_ref[...] = jnp.zeros_like(out_ref)

    x = x_ref[...]
    vs = vs_ref[...]
    nh = jax.lax.dot_general(
        vs, x, (((1,), (1,)), ((), ())), preferred_element_type=jnp.float32
    )
    p = jnp.exp(nh).astype(jnp.bfloat16)

    b = batch_ref[0, 0, :]
    g = jax.lax.broadcasted_iota(jnp.int32, (NG, BLK), 0)
    onehot = (g == b[None, :]).astype(jnp.bfloat16)

    for grp in range(BUMP_STEPS // 4):
        q = jnp.concatenate(
            [
                1.0 / (p + jnp.bfloat16(_E[s]))
                for s in range(4 * grp, 4 * grp + 4)
            ],
            axis=0,
        )
        contrib = jax.lax.dot_general(
            q, onehot, (((1,), (1,)), ((), ())),
            preferred_element_type=jnp.float32,
        )
        out_ref[pl.ds(128 * grp, 128), :] += contrib


def _sc_counts_call(batch_pad, npad_total, nw, lanes):
    chunk = npad_total // nw
    nsteps = chunk // lanes
    nslots = NG + 2

    mesh = plsc.VectorSubcoreMesh(core_axis_name="c", subcore_axis_name="s")

    @functools.partial(
        pl.kernel,
        mesh=mesh,
        out_type=jax.ShapeDtypeStruct((nw, NG * lanes), jnp.int32),
        scratch_types=[
            pltpu.VMEM((chunk,), jnp.int32),
            pltpu.VMEM((nslots * lanes,), jnp.int32),
        ],
    )
    def sc_counts(batch_hbm, out_hbm, b_v, hist_v):
        nc = lax.axis_size("c")
        wid = lax.axis_index("s") * nc + lax.axis_index("c")
        base = wid * chunk
        pltpu.sync_copy(batch_hbm.at[pl.ds(base, chunk)], b_v)
        zeros = jnp.zeros((lanes,), jnp.int32)
        for j in range(nslots):
            hist_v[pl.ds(j * lanes, lanes)] = zeros

        g_lo = b_v[pl.ds(0, lanes)][0]
        g_hi = b_v[pl.ds(chunk - lanes, lanes)][lanes - 1]

        def g_body(g, carry):
            def c_body(j, acc):
                eq = b_v[pl.ds(j * lanes, lanes)] == g
                return acc + jnp.where(eq, 1, 0)

            acc = lax.fori_loop(
                0, nsteps, c_body, jnp.zeros((lanes,), jnp.int32)
            )
            hist_v[pl.ds(g * lanes, lanes)] = acc
            return carry

        lax.fori_loop(g_lo, g_hi + 1, g_body, 0)
        pltpu.sync_copy(hist_v.at[pl.ds(0, NG * lanes)], out_hbm.at[wid])

    return sc_counts(batch_pad)


def kernel(x, batch, num_graphs, v):
    del num_graphs
    n = x.shape[0]
    nblocks = (n + BLK - 1) // BLK
    npad = nblocks * BLK - n
    if npad:
        x = jnp.pad(x, ((0, npad), (0, 0)))
        batch = jnp.pad(batch, (0, npad), constant_values=NG)
    batch3 = batch.reshape(nblocks, 1, BLK)
    x = x.astype(jnp.bfloat16)
    vs = (SCALE * v).astype(jnp.bfloat16)

    info = plsc.get_sparse_core_info()
    nw = info.num_cores * info.num_subcores
    lanes = info.num_lanes
    tile_cnt = _sc_counts_call(batch, nblocks * BLK, nw, lanes)

    out = pl.pallas_call(
        _fused,
        grid=(nblocks,),
        in_specs=[
            pl.BlockSpec((1, 1, BLK), lambda i: (i, 0, 0)),
            pl.BlockSpec((BLK, NUM_FEATURES), lambda i: (i, 0)),
            pl.BlockSpec((NUM_THETAS, NUM_FEATURES), lambda i: (0, 0)),
        ],
        out_specs=pl.BlockSpec((ST, NG), lambda i: (0, 0)),
        out_shape=jax.ShapeDtypeStruct((ST, NG), jnp.float32),
        compiler_params=pltpu.CompilerParams(
            dimension_semantics=("arbitrary",),
        ),
    )(batch3, x, vs)

    counts = jnp.sum(
        tile_cnt.reshape(nw, NG, lanes), axis=(0, 2)
    ).astype(jnp.float32)

    lin = jnp.asarray(_LIN)
    e_col = jnp.repeat(jnp.exp(SCALE * lin), NUM_THETAS)[:, None]
    c_row = jnp.repeat(jax.nn.sigmoid(SCALE * (lin - R)), NUM_THETAS)[None, :]
    res = (out * e_col).T - counts[:, None] * c_row
    return res.reshape(NG, BUMP_STEPS, NUM_THETAS)


# speedup vs baseline: 1.4358x; 1.4358x over previous
import jax
import jax.numpy as jnp
import numpy as np
from jax.experimental import pallas as pl
from jax.experimental.pallas import tpu as pltpu

NUM_THETAS = 32
BUMP_STEPS = 32
NUM_FEATURES = 128
R = 1.1
SCALE = 8.0
NG = 64
BLK = 12544
ST = BUMP_STEPS * NUM_THETAS

_LIN = np.linspace(-R, R, BUMP_STEPS, dtype=np.float32)
_E = [float(v) for v in np.exp(np.float64(SCALE) * _LIN)]


def _fused(batch_ref, x_ref, vs_ref, out_ref, cnt_ref):
    i = pl.program_id(0)

    @pl.when(i == 0)
    def _init():
        out_ref[...] = jnp.zeros_like(out_ref)
        cnt_ref[...] = jnp.zeros_like(cnt_ref)

    x = x_ref[...]
    vs = vs_ref[...]
    nh = jax.lax.dot_general(
        vs, x, (((1,), (1,)), ((), ())), preferred_element_type=jnp.float32
    )
    p = jnp.exp(nh).astype(jnp.bfloat16)

    b = batch_ref[0, 0, :]
    g = jax.lax.broadcasted_iota(jnp.int32, (NG, BLK), 0)
    onehot = (g == b[None, :]).astype(jnp.bfloat16)

    for grp in range(BUMP_STEPS // 4):
        q = jnp.concatenate(
            [
                1.0 / (p + jnp.bfloat16(_E[s]))
                for s in range(4 * grp, 4 * grp + 4)
            ],
            axis=0,
        )
        contrib = jax.lax.dot_general(
            q, onehot, (((1,), (1,)), ((), ())),
            preferred_element_type=jnp.float32,
        )
        out_ref[pl.ds(128 * grp, 128), :] += contrib

    ones = jnp.ones((8, BLK), dtype=jnp.bfloat16)
    cnt = jax.lax.dot_general(
        ones, onehot, (((1,), (1,)), ((), ())),
        preferred_element_type=jnp.float32,
    )
    cnt_ref[...] += cnt


def kernel(x, batch, num_graphs, v):
    del num_graphs
    n = x.shape[0]
    nblocks = (n + BLK - 1) // BLK
    npad = nblocks * BLK - n
    if npad:
        x = jnp.pad(x, ((0, npad), (0, 0)))
        batch = jnp.pad(batch, (0, npad), constant_values=NG)
    batch3 = batch.reshape(nblocks, 1, BLK)
    x = x.astype(jnp.bfloat16)
    vs = (SCALE * v).astype(jnp.bfloat16)

    out, cnt = pl.pallas_call(
        _fused,
        grid=(nblocks,),
        in_specs=[
            pl.BlockSpec((1, 1, BLK), lambda i: (i, 0, 0)),
            pl.BlockSpec((BLK, NUM_FEATURES), lambda i: (i, 0)),
            pl.BlockSpec((NUM_THETAS, NUM_FEATURES), lambda i: (0, 0)),
        ],
        out_specs=[
            pl.BlockSpec((ST, NG), lambda i: (0, 0)),
            pl.BlockSpec((8, NG), lambda i: (0, 0)),
        ],
        out_shape=[
            jax.ShapeDtypeStruct((ST, NG), jnp.float32),
            jax.ShapeDtypeStruct((8, NG), jnp.float32),
        ],
        compiler_params=pltpu.CompilerParams(
            dimension_semantics=("arbitrary",),
        ),
    )(batch3, x, vs)

    lin = jnp.asarray(_LIN)
    e_col = jnp.repeat(jnp.exp(SCALE * lin), NUM_THETAS)[:, None]
    c_row = jnp.repeat(jax.nn.sigmoid(SCALE * (lin - R)), NUM_THETAS)[None, :]
    res = (out * e_col).T - cnt[0][None, :].T * c_row
    return res.reshape(NG, BUMP_STEPS, NUM_THETAS)
